# baseline (speedup 1.0000x reference)
import functools

import jax
import jax.numpy as jnp
from jax import lax
from jax.experimental import pallas as pl
from jax.experimental.pallas import tpu as pltpu
from jax.experimental.pallas import tpu_sc as plsc

N = 10000
E = 320000
D = 128
DH = D // 2

NC = 2
NS = 16
CHUNK = 128
NW = NC * NS
EROWS = E // CHUNK
ROWS_W = EROWS // NW
XTRA = EROWS - ROWS_W * NW
XPC = XTRA // NC
NPAD = 10240
RPS = NPAD // NS
TRPS = N // NS
PHROWS = 13
NBUF = 3


def _sc_aggregate(feat, edges):
    mesh = plsc.VectorSubcoreMesh(core_axis_name="c", subcore_axis_name="s")

    @functools.partial(
        pl.kernel,
        out_type=(
            jax.ShapeDtypeStruct((NC, NPAD, D), jnp.float32),
            jax.ShapeDtypeStruct((NW, NPAD), jnp.float32),
        ),
        mesh=mesh,
        compiler_params=pltpu.CompilerParams(use_tc_tiling_on_sc=False,
                                             needs_layout_passes=False),
        scratch_types=[
            pltpu.VMEM_SHARED((N, DH), jnp.float32),
            pltpu.VMEM_SHARED((NPAD, DH), jnp.float32),
            pltpu.VMEM((PHROWS * CHUNK,), jnp.int32),
            pltpu.VMEM((PHROWS * CHUNK,), jnp.int32),
            pltpu.VMEM((PHROWS, CHUNK), jnp.int32),
            pltpu.VMEM((NBUF, CHUNK, DH), jnp.float32),
            pltpu.VMEM((NPAD,), jnp.float32),
            pltpu.SemaphoreType.DMA((NBUF,)),
            pltpu.SemaphoreType.DMA((NBUF,)),
        ],
    )
    def k(feat_hbm, edge_hbm, sum_hbm, deg_hbm,
          tab_sh, acc_sh, src_v, dst_v, dst2_v, rows_v, deg_t, gsem, ssem):
        c = lax.axis_index("c")
        s = lax.axis_index("s")
        w = c * NS + s
        base = s * RPS
        tbase = s * TRPS
        ebase = w * ROWS_W

        zeros16 = jnp.zeros((16,), jnp.float32)
        ones16 = jnp.ones((16,), jnp.float32)

        @pl.loop(0, NPAD // 16)
        def _(i):
            deg_t[pl.ds(i * 16, 16)] = zeros16

        def histo(idx_row):
            for q in range(CHUNK // 16):
                plsc.addupdate_scatter(
                    deg_t, [idx_row[pl.ds(q * 16, 16)]], ones16)

        def process_phase(h, nrows):
            hs, ss = {}, {}

            def scat(rr):
                hs[rr].wait()
                ss[rr] = pltpu.async_copy(
                    rows_v.at[rr % NBUF], acc_sh.at[dst2_v.at[rr]],
                    ssem.at[rr % NBUF], add=True)
                if h == 0:
                    histo(dst2_v.at[rr])

            for r in range(nrows):
                if r >= NBUF:
                    ss[r - NBUF].wait()
                hs[r] = pltpu.async_copy(
                    tab_sh.at[src_v.at[pl.ds(r * CHUNK, CHUNK)]],
                    rows_v.at[r % NBUF], gsem.at[r % NBUF])
                if r >= NBUF - 1:
                    scat(r - (NBUF - 1))
            for rr in range(max(0, nrows - (NBUF - 1)), nrows):
                scat(rr)
            for rr in range(max(0, nrows - NBUF), nrows):
                ss[rr].wait()

        for h in (0, 1):
            pltpu.sync_copy(
                feat_hbm.at[pl.ds(tbase, TRPS), pl.ds(h * DH, DH)],
                tab_sh.at[pl.ds(tbase, TRPS)])

            @pl.loop(0, CHUNK)
            def _(i):
                @pl.loop(0, DH // 16)
                def _(j):
                    rows_v[0, i, pl.ds(j * 16, 16)] = zeros16

            @pl.loop(0, RPS // CHUNK)
            def _(i):
                pltpu.sync_copy(rows_v.at[0],
                                acc_sh.at[pl.ds(base + i * CHUNK, CHUNK)])

            plsc.subcore_barrier()

            @pl.loop(0, ROWS_W // PHROWS)
            def _(ph):
                e0 = (ebase + ph * PHROWS) * CHUNK
                pltpu.sync_copy(edge_hbm.at[0, pl.ds(e0, PHROWS * CHUNK)],
                                src_v)
                pltpu.sync_copy(edge_hbm.at[1, pl.ds(e0, PHROWS * CHUNK)],
                                dst_v)

                @pl.loop(0, PHROWS * CHUNK // 16)
                def _(q):
                    v = dst_v[pl.ds(q * 16, 16)]
                    dst2_v[q // (CHUNK // 16),
                           pl.ds((q % (CHUNK // 16)) * 16, 16)] = v

                process_phase(h, PHROWS)

            @pl.when(s < XPC)
            def _():
                e0 = (NW * ROWS_W + c * XPC + s) * CHUNK
                pltpu.sync_copy(edge_hbm.at[0, pl.ds(e0, CHUNK)],
                                src_v.at[pl.ds(0, CHUNK)])
                pltpu.sync_copy(edge_hbm.at[1, pl.ds(e0, CHUNK)],
                                dst_v.at[pl.ds(0, CHUNK)])

                @pl.loop(0, CHUNK // 16)
                def _(q):
                    dst2_v[0, pl.ds(q * 16, 16)] = dst_v[pl.ds(q * 16, 16)]

                xg = pltpu.async_copy(
                    tab_sh.at[src_v.at[pl.ds(0, CHUNK)]], rows_v.at[0],
                    gsem.at[0])
                xg.wait()
                xs = pltpu.async_copy(
                    rows_v.at[0], acc_sh.at[dst2_v.at[0]],
                    ssem.at[0], add=True)
                if h == 0:
                    histo(dst2_v.at[0])
                xs.wait()

            plsc.subcore_barrier()

            pltpu.sync_copy(
                acc_sh.at[pl.ds(base, RPS)],
                sum_hbm.at[c, pl.ds(base, RPS), pl.ds(h * DH, DH)])

        pltpu.sync_copy(deg_t, deg_hbm.at[w])

    return k(feat, edges)


BLK = 2048
NGRID = (N + BLK - 1) // BLK


def _hself_body(feat_ref, ws_ref, b_ref, out_ref):
    dn = (((1,), (1,)), ((), ()))
    out_ref[...] = lax.dot_general(
        feat_ref[...], ws_ref[...], dn,
        preferred_element_type=jnp.float32,
        precision=lax.Precision.HIGHEST) + b_ref[...]


def _combine_body(hself_ref, parts_ref, deg_ref, wn_ref, out_ref):
    summed = parts_ref[0] + parts_ref[1]
    deg = jnp.sum(deg_ref[...], axis=0)[:, None]
    h_neigh = summed / jnp.maximum(deg, 1.0)
    dn = (((1,), (1,)), ((), ()))
    out_ref[...] = hself_ref[...] + lax.dot_general(
        h_neigh, wn_ref[...], dn,
        preferred_element_type=jnp.float32,
        precision=lax.Precision.HIGHEST)


def _tc_hself(feat, W_self, bias):
    return pl.pallas_call(
        _hself_body,
        grid=(NGRID,),
        in_specs=[
            pl.BlockSpec((BLK, D), lambda i: (i, 0)),
            pl.BlockSpec((D, D), lambda i: (0, 0)),
            pl.BlockSpec((1, D), lambda i: (0, 0)),
        ],
        out_specs=pl.BlockSpec((BLK, D), lambda i: (i, 0)),
        out_shape=jax.ShapeDtypeStruct((N, D), jnp.float32),
    )(feat, W_self, bias)


def _tc_combine(hself, parts, degp, W_neigh):
    return pl.pallas_call(
        _combine_body,
        grid=(NGRID,),
        in_specs=[
            pl.BlockSpec((BLK, D), lambda i: (i, 0)),
            pl.BlockSpec((NC, BLK, D), lambda i: (0, i, 0)),
            pl.BlockSpec((NW, BLK), lambda i: (0, i)),
            pl.BlockSpec((D, D), lambda i: (0, 0)),
        ],
        out_specs=pl.BlockSpec((BLK, D), lambda i: (i, 0)),
        out_shape=jax.ShapeDtypeStruct((N, D), jnp.float32),
    )(hself, parts, degp, W_neigh)


def kernel(feat, edge_index, W_self, b_self, W_neigh, b_neigh):
    parts, degp = _sc_aggregate(feat, edge_index)
    bias = (b_self + b_neigh).reshape(1, D)
    hself = _tc_hself(feat, W_self, bias)
    return _tc_combine(hself, parts, degp, W_neigh)


# speedup vs baseline: 1.0808x; 1.0808x over previous
import functools

import jax
import jax.numpy as jnp
from jax import lax
from jax.experimental import pallas as pl
from jax.experimental.pallas import tpu as pltpu
from jax.experimental.pallas import tpu_sc as plsc

N = 10000
E = 320000
D = 128
DH = D // 2

NC = 2
NS = 16
CHUNK = 128
NW = NC * NS
EROWS = E // CHUNK
ROWS_W = EROWS // NW
XTRA = EROWS - ROWS_W * NW
XPC = XTRA // NC
NPAD = 10240
RPS = NPAD // NS
TRPS = N // NS
PHROWS = 13
NBUF = 3


def _sc_aggregate(feat, edges):
    mesh = plsc.VectorSubcoreMesh(core_axis_name="c", subcore_axis_name="s")

    @functools.partial(
        pl.kernel,
        out_type=(
            jax.ShapeDtypeStruct((NC, NPAD, D), jnp.float32),
            jax.ShapeDtypeStruct((NW, NPAD), jnp.float32),
        ),
        mesh=mesh,
        compiler_params=pltpu.CompilerParams(use_tc_tiling_on_sc=False,
                                             needs_layout_passes=False),
        scratch_types=[
            pltpu.VMEM_SHARED((N, DH), jnp.float32),
            pltpu.VMEM_SHARED((NPAD, DH), jnp.float32),
            pltpu.VMEM((2, PHROWS * CHUNK), jnp.int32),
            pltpu.VMEM((2, PHROWS * CHUNK), jnp.int32),
            pltpu.VMEM((NBUF, CHUNK, DH), jnp.float32),
            pltpu.VMEM((NPAD,), jnp.float32),
            pltpu.SemaphoreType.DMA((NBUF,)),
            pltpu.SemaphoreType.DMA((NBUF,)),
            pltpu.SemaphoreType.DMA,
        ],
    )
    def k(feat_hbm, edge_hbm, sum_hbm, deg_hbm,
          tab_sh, acc_sh, src_v, dst_v, rows_v, deg_t, gsem, ssem, msem):
        c = lax.axis_index("c")
        s = lax.axis_index("s")
        w = c * NS + s
        base = s * RPS
        tbase = s * TRPS
        ebase = w * ROWS_W

        zeros16 = jnp.zeros((16,), jnp.float32)
        ones16 = jnp.ones((16,), jnp.float32)

        @pl.loop(0, NPAD // 16)
        def _(i):
            deg_t[pl.ds(i * 16, 16)] = zeros16

        def histo(idx_row):
            for q in range(CHUNK // 16):
                plsc.addupdate_scatter(
                    deg_t, [idx_row[pl.ds(q * 16, 16)]], ones16)

        def process_phase(h, nrows, pb, prefetch):
            hs, ss = {}, {}

            def scat(rr):
                hs[rr].wait()
                ss[rr] = pltpu.async_copy(
                    rows_v.at[rr % NBUF],
                    acc_sh.at[dst_v.at[pb, pl.ds(rr * CHUNK, CHUNK)]],
                    ssem.at[rr % NBUF], add=True)
                if h == 0:
                    histo(dst_v.at[pb].at[pl.ds(rr * CHUNK, CHUNK)])

            pf = []
            for r in range(nrows):
                if r >= NBUF:
                    ss[r - NBUF].wait()
                hs[r] = pltpu.async_copy(
                    tab_sh.at[src_v.at[pb, pl.ds(r * CHUNK, CHUNK)]],
                    rows_v.at[r % NBUF], gsem.at[r % NBUF])
                if r == 0 and prefetch is not None:
                    pf = prefetch()
                if r >= NBUF - 1:
                    scat(r - (NBUF - 1))
            for rr in range(max(0, nrows - (NBUF - 1)), nrows):
                scat(rr)
            for rr in range(max(0, nrows - NBUF), nrows):
                ss[rr].wait()
            return pf

        def load_idx(ph, pb):
            e0 = (ebase + ph * PHROWS) * CHUNK
            return [
                pltpu.async_copy(edge_hbm.at[0, pl.ds(e0, PHROWS * CHUNK)],
                                 src_v.at[pb], msem),
                pltpu.async_copy(edge_hbm.at[1, pl.ds(e0, PHROWS * CHUNK)],
                                 dst_v.at[pb], msem),
            ]

        NPH = ROWS_W // PHROWS

        for h in (0, 1):
            stg = pltpu.async_copy(
                feat_hbm.at[pl.ds(tbase, TRPS), pl.ds(h * DH, DH)],
                tab_sh.at[pl.ds(tbase, TRPS)], msem)

            @pl.loop(0, CHUNK)
            def _(i):
                @pl.loop(0, DH // 16)
                def _(j):
                    rows_v[0, i, pl.ds(j * 16, 16)] = zeros16

            @pl.loop(0, RPS // CHUNK)
            def _(i):
                pltpu.sync_copy(rows_v.at[0],
                                acc_sh.at[pl.ds(base + i * CHUNK, CHUNK)])

            pf = load_idx(0, 0)
            stg.wait()
            plsc.subcore_barrier()

            for ph in range(NPH):
                for d in pf:
                    d.wait()
                nxt = (lambda p=ph: (lambda: load_idx(p + 1, (p + 1) % 2)))                     if ph + 1 < NPH else (lambda: None)
                pf = process_phase(h, PHROWS, ph % 2, nxt())
                pf = pf or []

            @pl.when(s < XPC)
            def _():
                e0 = (NW * ROWS_W + c * XPC + s) * CHUNK
                pltpu.sync_copy(edge_hbm.at[0, pl.ds(e0, CHUNK)],
                                src_v.at[0, pl.ds(0, CHUNK)])
                pltpu.sync_copy(edge_hbm.at[1, pl.ds(e0, CHUNK)],
                                dst_v.at[0, pl.ds(0, CHUNK)])
                xg = pltpu.async_copy(
                    tab_sh.at[src_v.at[0, pl.ds(0, CHUNK)]], rows_v.at[0],
                    gsem.at[0])
                xg.wait()
                xs = pltpu.async_copy(
                    rows_v.at[0], acc_sh.at[dst_v.at[0, pl.ds(0, CHUNK)]],
                    ssem.at[0], add=True)
                if h == 0:
                    histo(dst_v.at[0].at[pl.ds(0, CHUNK)])
                xs.wait()

            plsc.subcore_barrier()

            pltpu.sync_copy(
                acc_sh.at[pl.ds(base, RPS)],
                sum_hbm.at[c, pl.ds(base, RPS), pl.ds(h * DH, DH)])

        pltpu.sync_copy(deg_t, deg_hbm.at[w])

    return k(feat, edges)


BLK = 2048
NGRID = (N + BLK - 1) // BLK


def _hself_body(feat_ref, ws_ref, b_ref, out_ref):
    dn = (((1,), (1,)), ((), ()))
    out_ref[...] = lax.dot_general(
        feat_ref[...], ws_ref[...], dn,
        preferred_element_type=jnp.float32,
        precision=lax.Precision.HIGHEST) + b_ref[...]


def _combine_body(hself_ref, parts_ref, deg_ref, wn_ref, out_ref):
    summed = parts_ref[0] + parts_ref[1]
    deg = jnp.sum(deg_ref[...], axis=0)[:, None]
    h_neigh = summed / jnp.maximum(deg, 1.0)
    dn = (((1,), (1,)), ((), ()))
    out_ref[...] = hself_ref[...] + lax.dot_general(
        h_neigh, wn_ref[...], dn,
        preferred_element_type=jnp.float32,
        precision=lax.Precision.HIGHEST)


def _tc_hself(feat, W_self, bias):
    return pl.pallas_call(
        _hself_body,
        grid=(NGRID,),
        in_specs=[
            pl.BlockSpec((BLK, D), lambda i: (i, 0)),
            pl.BlockSpec((D, D), lambda i: (0, 0)),
            pl.BlockSpec((1, D), lambda i: (0, 0)),
        ],
        out_specs=pl.BlockSpec((BLK, D), lambda i: (i, 0)),
        out_shape=jax.ShapeDtypeStruct((N, D), jnp.float32),
    )(feat, W_self, bias)


def _tc_combine(hself, parts, degp, W_neigh):
    return pl.pallas_call(
        _combine_body,
        grid=(NGRID,),
        in_specs=[
            pl.BlockSpec((BLK, D), lambda i: (i, 0)),
            pl.BlockSpec((NC, BLK, D), lambda i: (0, i, 0)),
            pl.BlockSpec((NW, BLK), lambda i: (0, i)),
            pl.BlockSpec((D, D), lambda i: (0, 0)),
        ],
        out_specs=pl.BlockSpec((BLK, D), lambda i: (i, 0)),
        out_shape=jax.ShapeDtypeStruct((N, D), jnp.float32),
    )(hself, parts, degp, W_neigh)


def kernel(feat, edge_index, W_self, b_self, W_neigh, b_neigh):
    parts, degp = _sc_aggregate(feat, edge_index)
    bias = (b_self + b_neigh).reshape(1, D)
    hself = _tc_hself(feat, W_self, bias)
    return _tc_combine(hself, parts, degp, W_neigh)
